# baseline (speedup 1.0000x reference)
import functools

import jax
import jax.numpy as jnp
from jax import lax
from jax.experimental import pallas as pl
from jax.experimental.pallas import tpu as pltpu
from jax.experimental.pallas import tpu_sc as plsc

B = 16384
EMB_DIM = 64
FEAT_DIM = 128

NC = 2
NS = 16
NW = NC * NS
BPW = B // NW
CHUNK = 128
NCH = BPW // CHUNK

_sc_mesh = plsc.VectorSubcoreMesh(
    core_axis_name="c", subcore_axis_name="s", num_cores=NC, num_subcores=NS
)


@functools.partial(
    pl.kernel,
    out_type=jax.ShapeDtypeStruct((2, B, EMB_DIM), jnp.float32),
    mesh=_sc_mesh,
    scratch_types=[
        pltpu.VMEM((NCH, CHUNK), jnp.int32),
        pltpu.VMEM((NCH, CHUNK), jnp.int32),
        pltpu.VMEM((BPW, EMB_DIM), jnp.float32),
        pltpu.VMEM((BPW, EMB_DIM), jnp.float32),
        pltpu.SemaphoreType.DMA,
    ],
    compiler_params=pltpu.CompilerParams(use_tc_tiling_on_sc=False),
)
def _sc_gather(user_emb, item_emb, ids3, out, idx_u, idx_i, rows_u, rows_i, sem):
    c = lax.axis_index("c")
    s = lax.axis_index("s")
    wid = s * NC + c
    base = wid * BPW

    pltpu.sync_copy(ids3.at[0, wid], idx_u)
    pltpu.sync_copy(ids3.at[1, wid], idx_i)

    copies = []
    for j in range(NCH):
        copies.append(
            pltpu.async_copy(
                user_emb.at[idx_u.at[j]], rows_u.at[pl.ds(j * CHUNK, CHUNK)], sem
            )
        )
        copies.append(
            pltpu.async_copy(
                item_emb.at[idx_i.at[j]], rows_i.at[pl.ds(j * CHUNK, CHUNK)], sem
            )
        )
    for cp in copies:
        cp.wait()

    pltpu.sync_copy(rows_u, out.at[0, pl.ds(base, BPW)])
    pltpu.sync_copy(rows_i, out.at[1, pl.ds(base, BPW)])


def _tc_body(g_ref, f_ref, w_ref, o_ref):
    o_ref[0] = g_ref[0] + jnp.dot(
        f_ref[0], w_ref[0], preferred_element_type=jnp.float32
    )


_BM = 2048


def _tc_call(gathered, feats, projt):
    return pl.pallas_call(
        _tc_body,
        grid=(2, B // _BM),
        in_specs=[
            pl.BlockSpec((1, _BM, EMB_DIM), lambda g, b: (g, b, 0)),
            pl.BlockSpec((1, _BM, FEAT_DIM), lambda g, b: (g, b, 0)),
            pl.BlockSpec((1, FEAT_DIM, EMB_DIM), lambda g, b: (g, 0, 0)),
        ],
        out_specs=pl.BlockSpec((1, _BM, EMB_DIM), lambda g, b: (g, b, 0)),
        out_shape=jax.ShapeDtypeStruct((2, B, EMB_DIM), jnp.float32),
    )(gathered, feats, projt)


def kernel(user_ids, item_ids, user_feats, item_feats, user_emb, item_emb,
           user_feat_proj, item_feat_proj):
    ids3 = (
        jnp.stack([user_ids, item_ids])
        .astype(jnp.int32)
        .reshape(2, NW, NCH, CHUNK)
    )
    gathered = _sc_gather(user_emb, item_emb, ids3)
    feats = jnp.stack([user_feats, item_feats])
    projt = jnp.stack([user_feat_proj.T, item_feat_proj.T])
    return _tc_call(gathered, feats, projt)


# speedup vs baseline: 1.0099x; 1.0099x over previous
import functools

import jax
import jax.numpy as jnp
from jax import lax
from jax.experimental import pallas as pl
from jax.experimental.pallas import tpu as pltpu
from jax.experimental.pallas import tpu_sc as plsc

B = 16384
EMB_DIM = 64
FEAT_DIM = 128

NC = 2
NS = 16
NW = NC * NS
BPW = B // NW
CHUNK = 128
NCH = BPW // CHUNK

_sc_mesh = plsc.VectorSubcoreMesh(
    core_axis_name="c", subcore_axis_name="s", num_cores=NC, num_subcores=NS
)


@functools.partial(
    pl.kernel,
    out_type=jax.ShapeDtypeStruct((2, B, EMB_DIM), jnp.float32),
    mesh=_sc_mesh,
    scratch_types=[
        pltpu.VMEM((BPW,), jnp.int32),
        pltpu.VMEM((BPW,), jnp.int32),
        pltpu.VMEM((BPW, EMB_DIM), jnp.float32),
        pltpu.VMEM((BPW, EMB_DIM), jnp.float32),
        pltpu.SemaphoreType.DMA,
    ],
    compiler_params=pltpu.CompilerParams(use_tc_tiling_on_sc=False),
)
def _sc_gather(user_emb, item_emb, user_ids, item_ids, out,
               idx_u, idx_i, rows_u, rows_i, sem):
    c = lax.axis_index("c")
    s = lax.axis_index("s")
    wid = s * NC + c
    base = wid * BPW

    pltpu.sync_copy(user_ids.at[pl.ds(base, BPW)], idx_u)
    pltpu.sync_copy(item_ids.at[pl.ds(base, BPW)], idx_i)

    copies = []
    for j in range(NCH):
        sl = pl.ds(j * CHUNK, CHUNK)
        copies.append(pltpu.async_copy(user_emb.at[idx_u.at[sl]], rows_u.at[sl], sem))
        copies.append(pltpu.async_copy(item_emb.at[idx_i.at[sl]], rows_i.at[sl], sem))
    for cp in copies:
        cp.wait()

    pltpu.sync_copy(rows_u, out.at[0, pl.ds(base, BPW)])
    pltpu.sync_copy(rows_i, out.at[1, pl.ds(base, BPW)])


_BM = 2048
_DN = (((1,), (1,)), ((), ()))


def _tc_body(g_ref, fu_ref, fi_ref, wu_ref, wi_ref, o_ref):
    o_ref[0] = g_ref[0] + lax.dot_general(
        fu_ref[...], wu_ref[...], _DN, preferred_element_type=jnp.float32
    )
    o_ref[1] = g_ref[1] + lax.dot_general(
        fi_ref[...], wi_ref[...], _DN, preferred_element_type=jnp.float32
    )


def _tc_call(gathered, user_feats, item_feats, wu, wi):
    return pl.pallas_call(
        _tc_body,
        grid=(B // _BM,),
        in_specs=[
            pl.BlockSpec((2, _BM, EMB_DIM), lambda b: (0, b, 0)),
            pl.BlockSpec((_BM, FEAT_DIM), lambda b: (b, 0)),
            pl.BlockSpec((_BM, FEAT_DIM), lambda b: (b, 0)),
            pl.BlockSpec((EMB_DIM, FEAT_DIM), lambda b: (0, 0)),
            pl.BlockSpec((EMB_DIM, FEAT_DIM), lambda b: (0, 0)),
        ],
        out_specs=pl.BlockSpec((2, _BM, EMB_DIM), lambda b: (0, b, 0)),
        out_shape=jax.ShapeDtypeStruct((2, B, EMB_DIM), jnp.float32),
    )(gathered, user_feats, item_feats, wu, wi)


def kernel(user_ids, item_ids, user_feats, item_feats, user_emb, item_emb,
           user_feat_proj, item_feat_proj):
    gathered = _sc_gather(user_emb, item_emb,
                          user_ids.astype(jnp.int32), item_ids.astype(jnp.int32))
    return _tc_call(gathered, user_feats, item_feats,
                    user_feat_proj, item_feat_proj)


# speedup vs baseline: 1.0109x; 1.0011x over previous
import functools

import jax
import jax.numpy as jnp
from jax import lax
from jax.experimental import pallas as pl
from jax.experimental.pallas import tpu as pltpu
from jax.experimental.pallas import tpu_sc as plsc

B = 16384
EMB_DIM = 64
FEAT_DIM = 128

NC = 2
NS = 16
NW = NC * NS
BPW = B // NW
CHUNK = 128
NCH = BPW // CHUNK

_sc_mesh = plsc.VectorSubcoreMesh(
    core_axis_name="c", subcore_axis_name="s", num_cores=NC, num_subcores=NS
)


@functools.partial(
    pl.kernel,
    out_type=jax.ShapeDtypeStruct((2, B, EMB_DIM), jnp.float32),
    mesh=_sc_mesh,
    scratch_types=[
        pltpu.VMEM((BPW,), jnp.int32),
        pltpu.VMEM((BPW,), jnp.int32),
        pltpu.VMEM((BPW, EMB_DIM), jnp.float32),
        pltpu.VMEM((BPW, EMB_DIM), jnp.float32),
        pltpu.SemaphoreType.DMA,
    ],
    compiler_params=pltpu.CompilerParams(
        use_tc_tiling_on_sc=False, needs_layout_passes=True
    ),
)
def _sc_gather(user_emb, item_emb, user_ids, item_ids, out,
               idx_u, idx_i, rows_u, rows_i, sem):
    c = lax.axis_index("c")
    s = lax.axis_index("s")
    wid = s * NC + c
    base = wid * BPW

    pltpu.sync_copy(user_ids.at[pl.ds(base, BPW)], idx_u)
    pltpu.sync_copy(item_ids.at[pl.ds(base, BPW)], idx_i)

    copies = []
    for j in range(NCH):
        sl = pl.ds(j * CHUNK, CHUNK)
        copies.append(pltpu.async_copy(user_emb.at[idx_u.at[sl]], rows_u.at[sl], sem))
        copies.append(pltpu.async_copy(item_emb.at[idx_i.at[sl]], rows_i.at[sl], sem))
    for cp in copies:
        cp.wait()

    pltpu.sync_copy(rows_u, out.at[0, pl.ds(base, BPW)])
    pltpu.sync_copy(rows_i, out.at[1, pl.ds(base, BPW)])


_BM = 2048
_DN = (((1,), (1,)), ((), ()))


def _tc_body(g_ref, fu_ref, fi_ref, wu_ref, wi_ref, o_ref):
    o_ref[0] = g_ref[0] + lax.dot_general(
        fu_ref[...], wu_ref[...], _DN, preferred_element_type=jnp.float32
    )
    o_ref[1] = g_ref[1] + lax.dot_general(
        fi_ref[...], wi_ref[...], _DN, preferred_element_type=jnp.float32
    )


def _tc_call(gathered, user_feats, item_feats, wu, wi):
    return pl.pallas_call(
        _tc_body,
        grid=(B // _BM,),
        in_specs=[
            pl.BlockSpec((2, _BM, EMB_DIM), lambda b: (0, b, 0)),
            pl.BlockSpec((_BM, FEAT_DIM), lambda b: (b, 0)),
            pl.BlockSpec((_BM, FEAT_DIM), lambda b: (b, 0)),
            pl.BlockSpec((EMB_DIM, FEAT_DIM), lambda b: (0, 0)),
            pl.BlockSpec((EMB_DIM, FEAT_DIM), lambda b: (0, 0)),
        ],
        out_specs=pl.BlockSpec((2, _BM, EMB_DIM), lambda b: (0, b, 0)),
        out_shape=jax.ShapeDtypeStruct((2, B, EMB_DIM), jnp.float32),
    )(gathered, user_feats, item_feats, wu, wi)


def kernel(user_ids, item_ids, user_feats, item_feats, user_emb, item_emb,
           user_feat_proj, item_feat_proj):
    gathered = _sc_gather(user_emb, item_emb,
                          user_ids.astype(jnp.int32), item_ids.astype(jnp.int32))
    return _tc_call(gathered, user_feats, item_feats,
                    user_feat_proj, item_feat_proj)


# speedup vs baseline: 1.0148x; 1.0038x over previous
import functools

import jax
import jax.numpy as jnp
from jax import lax
from jax.experimental import pallas as pl
from jax.experimental.pallas import tpu as pltpu
from jax.experimental.pallas import tpu_sc as plsc

B = 16384
EMB_DIM = 64
FEAT_DIM = 128

NC = 2
NS = 16
NW = NC * NS
BPW = B // NW
CHUNK = 128
NCH = BPW // CHUNK

_sc_mesh = plsc.VectorSubcoreMesh(
    core_axis_name="c", subcore_axis_name="s", num_cores=NC, num_subcores=NS
)


@functools.partial(
    pl.kernel,
    out_type=jax.ShapeDtypeStruct((B, EMB_DIM), jnp.float32),
    mesh=_sc_mesh,
    scratch_types=[
        pltpu.VMEM((BPW,), jnp.int32),
        pltpu.VMEM((BPW, EMB_DIM), jnp.float32),
        pltpu.SemaphoreType.DMA,
    ],
    compiler_params=pltpu.CompilerParams(use_tc_tiling_on_sc=False),
)
def _sc_gather(emb, ids, out, idx_v, rows_v, sem):
    c = lax.axis_index("c")
    s = lax.axis_index("s")
    wid = s * NC + c
    base = wid * BPW

    pltpu.sync_copy(ids.at[pl.ds(base, BPW)], idx_v)
    copies = []
    for j in range(NCH):
        sl = pl.ds(j * CHUNK, CHUNK)
        copies.append(pltpu.async_copy(emb.at[idx_v.at[sl]], rows_v.at[sl], sem))
    for cp in copies:
        cp.wait()
    pltpu.sync_copy(rows_v, out.at[pl.ds(base, BPW)])


_DN = (((1,), (1,)), ((), ()))


def _tc_body(gu_ref, gi_ref, fu_ref, fi_ref, wu_ref, wi_ref, o_ref):
    o_ref[0] = gu_ref[...] + lax.dot_general(
        fu_ref[...], wu_ref[...], _DN, preferred_element_type=jnp.float32
    )
    o_ref[1] = gi_ref[...] + lax.dot_general(
        fi_ref[...], wi_ref[...], _DN, preferred_element_type=jnp.float32
    )


_BM = 2048


def _tc_call(gu, gi, user_feats, item_feats, wu, wi):
    return pl.pallas_call(
        _tc_body,
        grid=(B // _BM,),
        in_specs=[
            pl.BlockSpec((_BM, EMB_DIM), lambda b: (b, 0)),
            pl.BlockSpec((_BM, EMB_DIM), lambda b: (b, 0)),
            pl.BlockSpec((_BM, FEAT_DIM), lambda b: (b, 0)),
            pl.BlockSpec((_BM, FEAT_DIM), lambda b: (b, 0)),
            pl.BlockSpec((EMB_DIM, FEAT_DIM), lambda b: (0, 0)),
            pl.BlockSpec((EMB_DIM, FEAT_DIM), lambda b: (0, 0)),
        ],
        out_specs=pl.BlockSpec((2, _BM, EMB_DIM), lambda b: (0, b, 0)),
        out_shape=jax.ShapeDtypeStruct((2, B, EMB_DIM), jnp.float32),
    )(gu, gi, user_feats, item_feats, wu, wi)


def kernel(user_ids, item_ids, user_feats, item_feats, user_emb, item_emb,
           user_feat_proj, item_feat_proj):
    gu = _sc_gather(user_emb, user_ids.astype(jnp.int32))
    gi = _sc_gather(item_emb, item_ids.astype(jnp.int32))
    return _tc_call(gu, gi, user_feats, item_feats,
                    user_feat_proj, item_feat_proj)
